# baseline (speedup 1.0000x reference)
import functools

import jax
import jax.numpy as jnp
from jax import lax
from jax.experimental import pallas as pl
from jax.experimental.pallas import tpu as pltpu
from jax.experimental.pallas import tpu_sc as plsc

_RATIO = 0.75
_LANES = 16


@functools.cache
def _build_sc_call(T, B, C, remain_T):
    info = plsc.get_sparse_core_info()
    num_workers = info.num_cores * info.num_subcores
    chunk = 64
    nbuf = 3
    rows_pub = 512
    rows_con = remain_T - rows_pub
    SEG = T // _LANES

    assert B == _LANES and num_workers == 2 * B
    assert SEG & (SEG - 1) == 0 and T == SEG * _LANES
    assert rows_pub % chunk == 0 and rows_con % chunk == 0
    assert rows_pub % 8 == 0
    max_chunks = max(rows_pub, rows_con) // chunk

    mesh = plsc.VectorSubcoreMesh(core_axis_name="c", subcore_axis_name="s")

    @functools.partial(
        pl.kernel,
        mesh=mesh,
        compiler_params=pltpu.CompilerParams(needs_layout_passes=False),
        out_type=[
            jax.ShapeDtypeStruct((remain_T * B, C), jnp.float32),
            jax.ShapeDtypeStruct((B, remain_T), jnp.int32),
        ],
        scratch_types=[
            pltpu.VMEM((T,), jnp.int32),
            pltpu.VMEM((remain_T,), jnp.int32),
            pltpu.VMEM((remain_T,), jnp.int32),
            pltpu.VMEM((remain_T,), jnp.int32),
            pltpu.VMEM((remain_T,), jnp.int32),
            pltpu.VMEM((max_chunks, chunk), jnp.int32),
            [pltpu.VMEM((chunk, C), jnp.float32) for _ in range(nbuf)],
            pltpu.VMEM_SHARED((info.num_subcores // 2, rows_con), jnp.int32),
            pltpu.VMEM_SHARED((info.num_subcores // 2, rows_con), jnp.int32),
            pltpu.SemaphoreType.DMA,
            pltpu.SemaphoreType.DMA,
        ],
    )
    def shuffle(patches_hbm, selT_hbm, out_hbm, bwdT_hbm,
                pos_v, col_v, bwd_v, val_v, src_idx_v, dst_idx_v, bufs,
                val_sh, bwd_sh, gsem, osem):
        sid = lax.axis_index("s")
        wid = sid * info.num_cores + lax.axis_index("c")
        lane = lax.iota(jnp.int32, _LANES)
        col = wid % B

        def sorted_gather(nrows):
            gpc = chunk // _LANES

            def mk_idx(j, carry):
                v = val_v[pl.ds(j * _LANES, _LANES)]
                b = bwd_v[pl.ds(j * _LANES, _LANES)]
                src_idx_v[pl.ds(j * _LANES, _LANES)] = v * B + col
                dst_idx_v[j // gpc, pl.ds((j % gpc) * _LANES, _LANES)] = (
                    b * B + col)
                return carry
            lax.fori_loop(0, nrows // _LANES, mk_idx, 0)

            n_chunks = nrows // chunk

            def start_gather(k):
                return pltpu.async_copy(
                    patches_hbm.at[src_idx_v.at[pl.ds(k * chunk, chunk)]],
                    bufs[k % nbuf], gsem)

            def start_put(k):
                return pltpu.async_copy(
                    bufs[k % nbuf], out_hbm.at[dst_idx_v.at[k]], osem)

            puts = [None] * n_chunks
            gets = [None] * n_chunks
            for k in range(min(nbuf, n_chunks)):
                gets[k] = start_gather(k)
            for k in range(n_chunks):
                gets[k].wait()
                puts[k] = start_put(k)
                if k >= 1 and k - 1 + nbuf < n_chunks:
                    puts[k - 1].wait()
                    puts[k - 1] = None
                    gets[k - 1 + nbuf] = start_gather(k - 1 + nbuf)
            for p in puts:
                if p is not None:
                    p.wait()

        @pl.when(wid < B)
        def _argsort():
            pltpu.sync_copy(selT_hbm.at[wid], col_v)

            def init(j, carry):
                pos_v[pl.ds(j * _LANES, _LANES)] = jnp.full(
                    (_LANES,), -1, jnp.int32)
                return carry
            lax.fori_loop(0, T // _LANES, init, 0)

            def scatter_pos(j, carry):
                vals = col_v[pl.ds(j * _LANES, _LANES)]
                slot = (vals & (SEG - 1)) * _LANES + (vals // SEG)
                plsc.store_scatter(pos_v, [slot], j * _LANES + lane)
                return carry
            lax.fori_loop(0, remain_T // _LANES, scatter_pos, 0)

            def count(j, pc):
                pv = pos_v[pl.ds(j * _LANES, _LANES)]
                return pc + (pv >= 0).astype(jnp.int32)
            pc = lax.fori_loop(0, SEG, count,
                               jnp.zeros((_LANES,), jnp.int32))
            base = plsc.cumsum(pc) - pc

            def emit(j, b):
                pv = pos_v[pl.ds(j * _LANES, _LANES)]
                present = pv >= 0
                plsc.store_scatter(bwd_v, [b], pv, mask=present)
                plsc.store_scatter(val_v, [b], lane * SEG + j, mask=present)
                return b + present.astype(jnp.int32)
            lax.fori_loop(0, SEG, emit, base)

            pltpu.sync_copy(bwd_v, bwdT_hbm.at[wid])
            pltpu.sync_copy(val_v.at[pl.ds(rows_pub, rows_con)],
                            val_sh.at[sid])
            pltpu.sync_copy(bwd_v.at[pl.ds(rows_pub, rows_con)],
                            bwd_sh.at[sid])

        plsc.subcore_barrier()

        @pl.when(wid < B)
        def _gather_low():
            sorted_gather(rows_pub)

        @pl.when(wid >= B)
        def _gather_high():
            pltpu.sync_copy(val_sh.at[sid - info.num_subcores // 2],
                            val_v.at[pl.ds(0, rows_con)])
            pltpu.sync_copy(bwd_sh.at[sid - info.num_subcores // 2],
                            bwd_v.at[pl.ds(0, rows_con)])
            sorted_gather(rows_con)

    return shuffle


def kernel(patches, forward_indexes):
    T, B, C = patches.shape
    remain_T = int(T * (1 - _RATIO))
    sel = forward_indexes[:remain_T]
    call = _build_sc_call(T, B, C, remain_T)
    out_flat, bwdT = call(patches.reshape(T * B, C), sel.T)
    return out_flat.reshape(remain_T, B, C), sel, bwdT.T


# speedup vs baseline: 1.0825x; 1.0825x over previous
import functools

import jax
import jax.numpy as jnp
from jax import lax
from jax.experimental import pallas as pl
from jax.experimental.pallas import tpu as pltpu
from jax.experimental.pallas import tpu_sc as plsc

_RATIO = 0.75
_LANES = 16


@functools.cache
def _build_sc_call(T, B, C, remain_T):
    info = plsc.get_sparse_core_info()
    num_workers = info.num_cores * info.num_subcores
    N = remain_T * B
    chunk = 64
    nbuf = 3
    rows_bwd_w = 320
    rows_big_w = (N - B * rows_bwd_w) // (num_workers - B)

    assert B == _LANES and num_workers == 2 * B
    assert rows_bwd_w % chunk == 0 and rows_big_w % chunk == 0
    assert B * rows_bwd_w + (num_workers - B) * rows_big_w == N
    assert T % _LANES == 0 and remain_T % _LANES == 0
    max_rows_w = max(rows_bwd_w, rows_big_w)

    mesh = plsc.VectorSubcoreMesh(core_axis_name="c", subcore_axis_name="s")

    @functools.partial(
        pl.kernel,
        mesh=mesh,
        compiler_params=pltpu.CompilerParams(needs_layout_passes=False),
        out_type=[
            jax.ShapeDtypeStruct((N, C), jnp.float32),
            jax.ShapeDtypeStruct((B, remain_T), jnp.int32),
        ],
        scratch_types=[
            pltpu.VMEM((max_rows_w,), jnp.int32),
            pltpu.VMEM((max_rows_w,), jnp.int32),
            [pltpu.VMEM((chunk, C), jnp.float32) for _ in range(nbuf)],
            pltpu.VMEM((T,), jnp.int32),
            pltpu.VMEM((remain_T,), jnp.int32),
            pltpu.VMEM((remain_T,), jnp.int32),
            pltpu.SemaphoreType.DMA,
            pltpu.SemaphoreType.DMA,
        ],
    )
    def shuffle(patches_hbm, sel_flat_hbm, selT_hbm, out_hbm, bwdT_hbm,
                sel_v, idx_v, bufs, pos_v, col_v, bwd_v, gsem, osem):
        wid = lax.axis_index("s") * info.num_cores + lax.axis_index("c")
        lane = lax.iota(jnp.int32, _LANES)

        def gather_rows(base, nrows):
            pltpu.sync_copy(sel_flat_hbm.at[pl.ds(base, nrows)], sel_v.at[pl.ds(0, nrows)])

            def mk_idx(j, carry):
                s = sel_v[pl.ds(j * _LANES, _LANES)]
                idx_v[pl.ds(j * _LANES, _LANES)] = s * B + lane
                return carry
            lax.fori_loop(0, nrows // _LANES, mk_idx, 0)

            n_chunks = nrows // chunk

            def start_gather(k):
                return pltpu.async_copy(
                    patches_hbm.at[idx_v.at[pl.ds(k * chunk, chunk)]],
                    bufs[k % nbuf], gsem)

            def start_put(k):
                return pltpu.async_copy(
                    bufs[k % nbuf], out_hbm.at[pl.ds(base + k * chunk, chunk)],
                    osem)

            gets = [None] * n_chunks
            puts = [None] * n_chunks
            for k in range(min(nbuf, n_chunks)):
                gets[k] = start_gather(k)
            for k in range(n_chunks):
                gets[k].wait()
                puts[k] = start_put(k)
                if k >= 1 and k - 1 + nbuf < n_chunks:
                    puts[k - 1].wait()
                    puts[k - 1] = None
                    gets[k - 1 + nbuf] = start_gather(k - 1 + nbuf)
            for p in puts:
                if p is not None:
                    p.wait()

        @pl.when(wid >= B)
        def _big():
            gather_rows(B * rows_bwd_w + (wid - B) * rows_big_w, rows_big_w)

        SEG = T // _LANES

        @pl.when(wid < B)
        def _small_and_backward():
            gather_rows(wid * rows_bwd_w, rows_bwd_w)
            pltpu.sync_copy(selT_hbm.at[wid], col_v)

            def init(c, carry):
                pos_v[pl.ds(c * _LANES, _LANES)] = jnp.full(
                    (_LANES,), -1, jnp.int32)
                return carry
            lax.fori_loop(0, T // _LANES, init, 0)

            def scatter_pos(c, carry):
                vals = col_v[pl.ds(c * _LANES, _LANES)]
                slot = (vals & (SEG - 1)) * _LANES + (vals // SEG)
                plsc.store_scatter(pos_v, [slot], c * _LANES + lane)
                return carry
            lax.fori_loop(0, remain_T // _LANES, scatter_pos, 0)

            def count(j, pc):
                pv = pos_v[pl.ds(j * _LANES, _LANES)]
                return pc + (pv >= 0).astype(jnp.int32)
            pc = lax.fori_loop(0, SEG, count,
                               jnp.zeros((_LANES,), jnp.int32))
            base = plsc.cumsum(pc) - pc

            def emit(j, b):
                pv = pos_v[pl.ds(j * _LANES, _LANES)]
                present = pv >= 0
                plsc.store_scatter(bwd_v, [b], pv, mask=present)
                return b + present.astype(jnp.int32)
            lax.fori_loop(0, SEG, emit, base)

            pltpu.sync_copy(bwd_v, bwdT_hbm.at[wid])

    return shuffle


def kernel(patches, forward_indexes):
    T, B, C = patches.shape
    remain_T = int(T * (1 - _RATIO))
    sel = forward_indexes[:remain_T]
    call = _build_sc_call(T, B, C, remain_T)
    out_flat, bwdT = call(
        patches.reshape(T * B, C),
        sel.reshape(remain_T * B),
        sel.T,
    )
    return out_flat.reshape(remain_T, B, C), sel, bwdT.T
